# baseline (speedup 1.0000x reference)
import jax
import jax.numpy as jnp
from jax import lax
from jax.experimental import pallas as pl
from jax.experimental.pallas import tpu as pltpu
from jax.experimental.pallas import tpu_sc as plsc

D_MODEL = 64
SCALE = 8.0

NC = 2
NS = 16
NW = NC * NS

BB = 128
LANES = 16
NGB = 4
NTB = 2


def _body(xt_hbm, t128_hbm, out_hbm, idx_v, r0, r1, r2, r3, t0, t1,
          g0, g1, g2, g3, o0, o1):
    seq = xt_hbm.shape[0]
    wid = lax.axis_index("s") * NC + lax.axis_index("c")
    bcol = pl.multiple_of(wid * BB, BB)

    pltpu.sync_copy(xt_hbm.at[:, pl.ds(bcol, BB)], idx_v)

    rows = (r0, r1, r2, r3)
    trs = (t0, t1)
    gsems = (g0, g1, g2, g3)
    osems = (o0, o1)

    def gather_copy(s, k):
        return pltpu.make_async_copy(
            t128_hbm.at[idx_v.at[s]], rows[k], gsems[k])

    def out_copy(s, k):
        return pltpu.make_async_copy(
            trs[k], out_hbm.at[s, :, pl.ds(bcol, BB)], osems[k])

    iota = lax.iota(jnp.int32, LANES)
    fidx = [iota + j * LANES for j in range(D_MODEL // LANES)]

    zeros16 = jnp.full((LANES,), 0, jnp.int32)
    ones16 = jnp.full((LANES,), 1, jnp.int32)

    def transpose_scale(rbuf, tbuf):
        @plsc.parallel_loop(0, BB, 1, unroll=16, carry=zeros16)
        def trow(b, bvec):
            for j in range(D_MODEL // LANES):
                vals = rbuf[b, pl.ds(j * LANES, LANES)] * SCALE
                plsc.store_scatter(tbuf, [fidx[j], bvec], vals)
            return bvec + ones16

    for k in range(NGB):
        gather_copy(k, k).start()

    def step(su, carry):
        for par in range(NGB):
            s = NGB * su + par
            gather_copy(s, par).wait()

            @pl.when(s >= NTB)
            def _():
                out_copy(s - NTB, par % NTB).wait()

            transpose_scale(rows[par], trs[par % NTB])

            @pl.when(s + NGB < seq)
            def _():
                gather_copy(s + NGB, par).start()

            out_copy(s, par % NTB).start()
        return carry

    lax.fori_loop(0, seq // NGB, step, 0)
    out_copy(seq - 2, 0).wait()
    out_copy(seq - 1, 1).wait()


def kernel(x, table):
    b_total, seq = x.shape
    xt = x.T.astype(jnp.int32)
    t128 = jnp.pad(table, ((0, 0), (0, 64)))
    mesh = plsc.VectorSubcoreMesh(core_axis_name="c", subcore_axis_name="s")
    out_phys = pl.kernel(
        _body,
        mesh=mesh,
        compiler_params=pltpu.CompilerParams(
            use_tc_tiling_on_sc=True, needs_layout_passes=False,
            disable_bounds_checks=True),
        out_type=jax.ShapeDtypeStruct((seq, D_MODEL, b_total), jnp.float32),
        scratch_types=[
            pltpu.VMEM((seq, BB), jnp.int32),
            pltpu.VMEM((BB, 128), jnp.float32),
            pltpu.VMEM((BB, 128), jnp.float32),
            pltpu.VMEM((BB, 128), jnp.float32),
            pltpu.VMEM((BB, 128), jnp.float32),
            pltpu.VMEM((D_MODEL, BB), jnp.float32),
            pltpu.VMEM((D_MODEL, BB), jnp.float32),
            pltpu.SemaphoreType.DMA,
            pltpu.SemaphoreType.DMA,
            pltpu.SemaphoreType.DMA,
            pltpu.SemaphoreType.DMA,
            pltpu.SemaphoreType.DMA,
            pltpu.SemaphoreType.DMA,
        ],
    )(xt, t128)
    return out_phys.transpose(2, 0, 1)


# speedup vs baseline: 1.0380x; 1.0380x over previous
import jax
import jax.numpy as jnp
from jax import lax
from jax.experimental import pallas as pl
from jax.experimental.pallas import tpu as pltpu
from jax.experimental.pallas import tpu_sc as plsc

D_MODEL = 64
SCALE = 8.0

NC = 2
NS = 16
NW = NC * NS

BB = 128
LANES = 16
NGB = 4
NTB = 2


def _body(xt_hbm, t128_hbm, out_hbm, idx_v, r0, r1, r2, r3, t0, t1,
          g0, g1, g2, g3, o0, o1):
    seq = xt_hbm.shape[0]
    wid = lax.axis_index("s") * NC + lax.axis_index("c")
    bcol = pl.multiple_of(wid * BB, BB)

    pltpu.sync_copy(xt_hbm.at[:, pl.ds(bcol, BB)], idx_v)

    rows = (r0, r1, r2, r3)
    trs = (t0, t1)
    gsems = (g0, g1, g2, g3)
    osems = (o0, o1)

    def gather_copy(s, k):
        return pltpu.make_async_copy(
            t128_hbm.at[idx_v.at[s]], rows[k], gsems[k])

    def out_copy(s, k):
        return pltpu.make_async_copy(
            trs[k], out_hbm.at[s, :, pl.ds(bcol, BB)], osems[k])

    iota = lax.iota(jnp.int32, LANES)
    fidx = [iota + j * LANES for j in range(D_MODEL // LANES)]

    zeros16 = jnp.full((LANES,), 0, jnp.int32)
    ones16 = jnp.full((LANES,), 1, jnp.int32)

    toks = [iota + g * LANES for g in range(BB // LANES)]

    def transpose_scale(rbuf, tbuf):
        @plsc.parallel_loop(0, D_MODEL, 1, unroll=8, carry=zeros16)
        def tcol(f, fvec):
            for g in range(BB // LANES):
                vals = plsc.load_gather(rbuf, [toks[g], fvec])
                tbuf[f, pl.ds(g * LANES, LANES)] = vals * SCALE
            return fvec + ones16

    for k in range(NGB):
        gather_copy(k, k).start()

    def step(su, carry):
        for par in range(NGB):
            s = NGB * su + par
            gather_copy(s, par).wait()

            @pl.when(s >= NTB)
            def _():
                out_copy(s - NTB, par % NTB).wait()

            transpose_scale(rows[par], trs[par % NTB])

            @pl.when(s + NGB < seq)
            def _():
                gather_copy(s + NGB, par).start()

            out_copy(s, par % NTB).start()
        return carry

    lax.fori_loop(0, seq // NGB, step, 0)
    out_copy(seq - 2, 0).wait()
    out_copy(seq - 1, 1).wait()


def kernel(x, table):
    b_total, seq = x.shape
    xt = x.T.astype(jnp.int32)
    t128 = jnp.pad(table, ((0, 0), (0, 64)))
    mesh = plsc.VectorSubcoreMesh(core_axis_name="c", subcore_axis_name="s")
    out_phys = pl.kernel(
        _body,
        mesh=mesh,
        compiler_params=pltpu.CompilerParams(
            use_tc_tiling_on_sc=True, needs_layout_passes=False,
            disable_bounds_checks=True),
        out_type=jax.ShapeDtypeStruct((seq, D_MODEL, b_total), jnp.float32),
        scratch_types=[
            pltpu.VMEM((seq, BB), jnp.int32),
            pltpu.VMEM((BB, 128), jnp.float32),
            pltpu.VMEM((BB, 128), jnp.float32),
            pltpu.VMEM((BB, 128), jnp.float32),
            pltpu.VMEM((BB, 128), jnp.float32),
            pltpu.VMEM((D_MODEL, BB), jnp.float32),
            pltpu.VMEM((D_MODEL, BB), jnp.float32),
            pltpu.SemaphoreType.DMA,
            pltpu.SemaphoreType.DMA,
            pltpu.SemaphoreType.DMA,
            pltpu.SemaphoreType.DMA,
            pltpu.SemaphoreType.DMA,
            pltpu.SemaphoreType.DMA,
        ],
    )(xt, t128)
    return out_phys.transpose(2, 0, 1)
